# Initial kernel scaffold; baseline (speedup 1.0000x reference)
import jax
import jax.numpy as jnp
from jax.experimental import pallas as pl


def kernel(edge_index, rel_type, norm, weight):
    raise NotImplementedError("write your pallas kernel here")


# speedup vs baseline: 3.9711x; 3.9711x over previous
import jax
import jax.numpy as jnp
from jax import lax
from jax.experimental import pallas as pl
from jax.experimental.pallas import tpu as pltpu
from jax.experimental.pallas import tpu_sc as plsc

N_NODES = 10000
N_EDGES = 320000
IN_FEAT = 10000
OUT_FEAT = 128
NUM_RELS = 16

NC = 2
NS = 16
NW = NC * NS
EPW = N_EDGES // NW
K = 80
NCHUNK = EPW // K
NZCHUNK = N_NODES // K


def _sc_kernel(embed, src2, rel3, norm2, dst3, part,
               idx_v, rd_v, norm_v, rows_v, acc):
    cid = lax.axis_index("c")
    sid = lax.axis_index("s")
    wid = cid * NS + sid

    pltpu.sync_copy(src2.at[wid], idx_v)
    pltpu.sync_copy(rel3.at[wid], rd_v)
    pltpu.sync_copy(norm2.at[wid], norm_v)

    def idxbody(r, carry):
        for m in range(K // 16):
            sl = pl.ds(r * K + 16 * m, 16)
            idx_v[sl] = rd_v[r, pl.ds(16 * m, 16)] * IN_FEAT + idx_v[sl]
        return carry
    lax.fori_loop(0, NCHUNK, idxbody, 0)

    pltpu.sync_copy(dst3.at[wid], rd_v)

    def zrow(r, carry):
        for j in range(OUT_FEAT // 16):
            rows_v[r, pl.ds(16 * j, 16)] = jnp.zeros((16,), jnp.float32)
        return carry
    lax.fori_loop(0, K, zrow, 0)
    nzc = (NZCHUNK - sid + NS - 1) // NS

    def zcopy(t, carry):
        j = sid + t * NS
        pltpu.sync_copy(rows_v, acc.at[pl.ds(j * K, K)])
        return carry
    lax.fori_loop(0, nzc, zcopy, 0)

    plsc.subcore_barrier()

    def chunk(ci, carry):
        pltpu.sync_copy(embed.at[idx_v.at[pl.ds(ci * K, K)]], rows_v)

        def scale(g, c2):
            nv = norm_v[pl.ds(ci * K + 16 * g, 16)]
            for l in range(16):
                e = 16 * g + l
                nb = nv[l]
                for j in range(OUT_FEAT // 16):
                    sl = pl.ds(16 * j, 16)
                    rows_v[e, sl] = rows_v[e, sl] * nb
            return c2
        lax.fori_loop(0, K // 16, scale, 0)

        pltpu.sync_copy(rows_v, acc.at[rd_v.at[ci]], add=True)
        return carry
    lax.fori_loop(0, NCHUNK, chunk, 0)

    plsc.subcore_barrier()

    def wcopy(t, carry):
        j = sid + t * NS
        sl = pl.ds(j * K, K)
        pltpu.sync_copy(acc.at[sl], rows_v)
        pltpu.sync_copy(rows_v, part.at[cid, sl])
        return carry
    lax.fori_loop(0, nzc, wcopy, 0)


@jax.jit
def _rgcn_sc(embed, src2, rel3, norm2, dst3):
    mesh = plsc.VectorSubcoreMesh(core_axis_name="c", subcore_axis_name="s")
    return pl.kernel(
        _sc_kernel,
        out_type=jax.ShapeDtypeStruct((NC, N_NODES, OUT_FEAT), jnp.float32),
        mesh=mesh,
        scratch_types=[
            pltpu.VMEM((EPW,), jnp.int32),
            pltpu.VMEM((NCHUNK, K), jnp.int32),
            pltpu.VMEM((EPW,), jnp.float32),
            pltpu.VMEM((K, OUT_FEAT), jnp.float32),
            pltpu.VMEM_SHARED((N_NODES, OUT_FEAT), jnp.float32),
        ],
    )(embed, src2, rel3, norm2, dst3)


def _add_body(a_ref, b_ref, o_ref):
    o_ref[...] = a_ref[...] + b_ref[...]


@jax.jit
def _combine(part):
    blk = 1000
    spec = pl.BlockSpec((blk, OUT_FEAT), lambda i: (i, 0))
    return pl.pallas_call(
        _add_body,
        out_shape=jax.ShapeDtypeStruct((N_NODES, OUT_FEAT), jnp.float32),
        grid=(N_NODES // blk,),
        in_specs=[spec, spec],
        out_specs=spec,
    )(part[0], part[1])


def kernel(edge_index, rel_type, norm, weight):
    src2 = edge_index[0].reshape(NW, EPW)
    dst3 = edge_index[1].reshape(NW, NCHUNK, K)
    rel3 = rel_type.reshape(NW, NCHUNK, K)
    norm2 = norm.reshape(NW, EPW)
    embed = weight.reshape(NUM_RELS * IN_FEAT, OUT_FEAT)
    part = _rgcn_sc(embed, src2, rel3, norm2, dst3)
    return _combine(part)
